# baseline (speedup 1.0000x reference)
import jax
import jax.numpy as jnp
from jax import lax
from jax.experimental import pallas as pl
from jax.experimental.pallas import tpu as pltpu
from jax.experimental.pallas import tpu_sc as plsc

NUM_CLASSES = 80
DIM = 256
N_TOKENS = 131072

NC = 2
NS = 16
L = 16
NW = NC * NS

TOK_PER_W = N_TOKENS // NW
GROUPS = TOK_PER_W // L
MAXB = GROUPS + 1
ROWS_W = TOK_PER_W + L

TC_BLK = 8192
TC_GRID = N_TOKENS // TC_BLK
PADC = 128


def _tc_expand(labels3, table_pad):
    def body(lab_ref, tab_ref, out_ref):
        lab = lab_ref[0, 0, :]
        onehot = (lab[:, None] == lax.broadcasted_iota(
            jnp.int32, (TC_BLK, PADC), 1)).astype(jnp.float32)
        out_ref[...] = jnp.dot(onehot, tab_ref[...],
                               preferred_element_type=jnp.float32)

    return pl.pallas_call(
        body,
        grid=(TC_GRID,),
        in_specs=[
            pl.BlockSpec((1, 1, TC_BLK), lambda i: (i, 0, 0)),
            pl.BlockSpec((PADC, DIM), lambda i: (0, 0)),
        ],
        out_specs=pl.BlockSpec((TC_BLK, DIM), lambda i: (i, 0)),
        out_shape=jax.ShapeDtypeStruct((N_TOKENS, DIM), jnp.float32),
        compiler_params=pltpu.CompilerParams(
            dimension_semantics=("arbitrary",)),
    )(labels3, table_pad)


def _sc_stage(labels3, tokens, pointer):
    mesh = plsc.VectorSubcoreMesh(core_axis_name="c", subcore_axis_name="s")

    @pl.kernel(
        out_type=(
            jax.ShapeDtypeStruct((NW, MAXB, L), jnp.int32),
            jax.ShapeDtypeStruct((NW, L), jnp.int32),
            jax.ShapeDtypeStruct((NW * ROWS_W, DIM), jnp.float32),
        ),
        mesh=mesh,
        compiler_params=pltpu.CompilerParams(needs_layout_passes=False),
        scratch_types=[
            pltpu.VMEM((GROUPS, L), jnp.int32),
            pltpu.VMEM((NUM_CLASSES,), jnp.int32),
            pltpu.VMEM((TOK_PER_W + L,), jnp.int32),
            pltpu.VMEM((MAXB, L), jnp.int32),
            pltpu.VMEM((L,), jnp.int32),
            pltpu.VMEM((2, L, DIM), jnp.float32),
            pltpu.SemaphoreType.DMA((2,)),
            pltpu.SemaphoreType.DMA((2,)),
        ],
    )
    def body(labels_hbm, tokens_hbm, ptr_hbm, idx_out, nb_out, rows_out,
             lab_v, ptr_v, idx_flat, idx2d, nb_v, tok_v, gsem, wsem):
        wid = lax.axis_index("s") * NC + lax.axis_index("c")
        pltpu.sync_copy(labels_hbm.at[wid], lab_v)
        pltpu.sync_copy(ptr_hbm, ptr_v)
        base = wid * TOK_PER_W
        rowbase = wid * ROWS_W
        iota = lax.iota(jnp.int32, L)

        def scan_body(g, n):
            lab16 = lab_v[g, :]
            inv = plsc.load_gather(ptr_v, [lab16]) == 0
            gidx = jnp.full((L,), base + g * L, jnp.int32) + iota
            plsc.store_compressed(idx_flat.at[pl.ds(n, L)], gidx, mask=inv)
            return n + jnp.sum(inv.astype(jnp.int32))

        n = lax.fori_loop(0, GROUPS, scan_body, 0)
        nb = (n + (L - 1)) >> 4
        nb_v[...] = jnp.full((L,), nb, jnp.int32)
        pltpu.sync_copy(nb_v, nb_out.at[wid])

        @pl.when(n > 0)
        def _stage():
            dup0 = plsc.load_gather(idx_flat, [jnp.zeros((L,), jnp.int32)])
            idx_flat[pl.ds(n, L)] = dup0

            def copy_body(r, _):
                idx2d[r, :] = idx_flat[pl.ds(r * L, L)]
                return 0

            lax.fori_loop(0, nb, copy_body, 0)
            pltpu.sync_copy(idx2d, idx_out.at[wid])

            def batch_body(b, _):
                s = b % 2

                @pl.when(b >= 2)
                def _reclaim():
                    pltpu.make_async_copy(
                        tok_v.at[s],
                        rows_out.at[pl.ds(rowbase, L)],
                        wsem.at[s]).wait()

                pltpu.make_async_copy(
                    tokens_hbm.at[idx2d.at[b]], tok_v.at[s],
                    gsem.at[s]).start()

                @pl.when(b >= 1)
                def _retire():
                    sp = (b - 1) % 2
                    pltpu.make_async_copy(
                        tokens_hbm.at[idx2d.at[b - 1]], tok_v.at[sp],
                        gsem.at[sp]).wait()
                    pltpu.make_async_copy(
                        tok_v.at[sp],
                        rows_out.at[pl.ds(rowbase + (b - 1) * L, L)],
                        wsem.at[sp]).start()
                return 0

            lax.fori_loop(0, nb, batch_body, 0)

            sl = (nb - 1) % 2
            pltpu.make_async_copy(
                tokens_hbm.at[idx2d.at[nb - 1]], tok_v.at[sl],
                gsem.at[sl]).wait()
            pltpu.make_async_copy(
                tok_v.at[sl],
                rows_out.at[pl.ds(rowbase + (nb - 1) * L, L)],
                wsem.at[sl]).start()
            for s in range(2):
                @pl.when(nb > s)
                def _drain(s=s):
                    pltpu.make_async_copy(
                        tok_v.at[s],
                        rows_out.at[pl.ds(rowbase, L)],
                        wsem.at[s]).wait()

    return body(labels3, tokens, pointer)


def _sc_apply(out_ref_arg, idx_out, nb_out, rows_out):
    mesh = plsc.VectorSubcoreMesh(core_axis_name="c", subcore_axis_name="s")

    @pl.kernel(
        out_type=(),
        mesh=mesh,
        compiler_params=pltpu.CompilerParams(needs_layout_passes=False),
        scratch_types=[
            pltpu.VMEM((MAXB, L), jnp.int32),
            pltpu.VMEM((L,), jnp.int32),
            pltpu.VMEM((2, L, DIM), jnp.float32),
            pltpu.SemaphoreType.DMA((2,)),
            pltpu.SemaphoreType.DMA((2,)),
        ],
    )
    def body(out_hbm, idx_hbm, nb_hbm, rows_hbm,
             idx2d, nb_v, tok_v, gsem, ssem):
        wid = lax.axis_index("s") * NC + lax.axis_index("c")
        rowbase = wid * ROWS_W
        pltpu.sync_copy(nb_hbm.at[wid], nb_v)
        nb = jnp.max(nb_v[...])

        @pl.when(nb > 0)
        def _apply():
            pltpu.sync_copy(idx_hbm.at[wid], idx2d)

            def batch_body(b, _):
                s = b % 2

                @pl.when(b >= 2)
                def _reclaim():
                    pltpu.make_async_copy(
                        tok_v.at[s], out_hbm.at[idx2d.at[b]],
                        ssem.at[s]).wait()

                pltpu.make_async_copy(
                    rows_hbm.at[pl.ds(rowbase + b * L, L)], tok_v.at[s],
                    gsem.at[s]).start()

                @pl.when(b >= 1)
                def _retire():
                    sp = (b - 1) % 2
                    pltpu.make_async_copy(
                        rows_hbm.at[pl.ds(rowbase + (b - 1) * L, L)],
                        tok_v.at[sp], gsem.at[sp]).wait()
                    pltpu.make_async_copy(
                        tok_v.at[sp], out_hbm.at[idx2d.at[b - 1]],
                        ssem.at[sp]).start()
                return 0

            lax.fori_loop(0, nb, batch_body, 0)

            sl = (nb - 1) % 2
            pltpu.make_async_copy(
                rows_hbm.at[pl.ds(rowbase + (nb - 1) * L, L)], tok_v.at[sl],
                gsem.at[sl]).wait()
            pltpu.make_async_copy(
                tok_v.at[sl], out_hbm.at[idx2d.at[nb - 1]],
                ssem.at[sl]).start()
            for s in range(2):
                @pl.when(nb > s)
                def _drain(s=s):
                    pltpu.make_async_copy(
                        tok_v.at[s], out_hbm.at[idx2d.at[0]],
                        ssem.at[s]).wait()

    body(out_ref_arg, idx_out, nb_out, rows_out)


def kernel(tokens, token_labels, buffer, pointer):
    labels = token_labels.astype(jnp.int32)
    table_pad = jnp.zeros((PADC, DIM), jnp.float32).at[:NUM_CLASSES].set(
        buffer[:, 0, :])
    staged = _sc_stage(labels.reshape(NW, GROUPS, L), tokens,
                       pointer.astype(jnp.int32))
    dense = _tc_expand(labels.reshape(TC_GRID, 1, TC_BLK), table_pad)
    out_ref = jax.new_ref(dense)
    _sc_apply(out_ref, *staged)
    return jax.freeze(out_ref)


# speedup vs baseline: 1.0310x; 1.0310x over previous
import jax
import jax.numpy as jnp
from jax import lax
from jax.experimental import pallas as pl
from jax.experimental.pallas import tpu as pltpu
from jax.experimental.pallas import tpu_sc as plsc

NUM_CLASSES = 80
DIM = 256
N_TOKENS = 131072

NC = 2
NS = 16
L = 16
NW = NC * NS

TOK_PER_W = N_TOKENS // NW
GROUPS = TOK_PER_W // L
MAXB = GROUPS + 1

TC_BLK = 8192
TC_GRID = N_TOKENS // TC_BLK
PADC = 128


def _tc_expand(labels3, table_pad):
    def body(lab_ref, tab_ref, out_ref):
        lab = lab_ref[0, 0, :]
        onehot = (lab[:, None] == lax.broadcasted_iota(
            jnp.int32, (TC_BLK, PADC), 1)).astype(jnp.float32)
        out_ref[...] = jnp.dot(onehot, tab_ref[...],
                               preferred_element_type=jnp.float32)

    return pl.pallas_call(
        body,
        grid=(TC_GRID,),
        in_specs=[
            pl.BlockSpec((1, 1, TC_BLK), lambda i: (i, 0, 0)),
            pl.BlockSpec((PADC, DIM), lambda i: (0, 0)),
        ],
        out_specs=pl.BlockSpec((TC_BLK, DIM), lambda i: (i, 0)),
        out_shape=jax.ShapeDtypeStruct((N_TOKENS, DIM), jnp.float32),
        compiler_params=pltpu.CompilerParams(
            dimension_semantics=("arbitrary",)),
    )(labels3, table_pad)


def _sc_scan(labels3, pointer):
    mesh = plsc.VectorSubcoreMesh(core_axis_name="c", subcore_axis_name="s")

    @pl.kernel(
        out_type=(
            jax.ShapeDtypeStruct((NW, MAXB, L), jnp.int32),
            jax.ShapeDtypeStruct((NW, L), jnp.int32),
        ),
        mesh=mesh,
        compiler_params=pltpu.CompilerParams(needs_layout_passes=False),
        scratch_types=[
            pltpu.VMEM((GROUPS, L), jnp.int32),
            pltpu.VMEM((NUM_CLASSES,), jnp.int32),
            pltpu.VMEM((TOK_PER_W + L,), jnp.int32),
            pltpu.VMEM((MAXB, L), jnp.int32),
            pltpu.VMEM((L,), jnp.int32),
        ],
    )
    def body(labels_hbm, ptr_hbm, idx_out, nb_out,
             lab_v, ptr_v, idx_flat, idx2d, nb_v):
        wid = lax.axis_index("s") * NC + lax.axis_index("c")
        pltpu.sync_copy(labels_hbm.at[wid], lab_v)
        pltpu.sync_copy(ptr_hbm, ptr_v)
        base = wid * TOK_PER_W
        iota = lax.iota(jnp.int32, L)

        def scan_body(g, n):
            lab16 = lab_v[g, :]
            inv = plsc.load_gather(ptr_v, [lab16]) == 0
            gidx = jnp.full((L,), base + g * L, jnp.int32) + iota
            plsc.store_compressed(idx_flat.at[pl.ds(n, L)], gidx, mask=inv)
            return n + jnp.sum(inv.astype(jnp.int32))

        n = lax.fori_loop(0, GROUPS, scan_body, 0)
        nb = (n + (L - 1)) >> 4
        nb_v[...] = jnp.full((L,), nb, jnp.int32)
        pltpu.sync_copy(nb_v, nb_out.at[wid])

        @pl.when(n > 0)
        def _emit():
            dup0 = plsc.load_gather(idx_flat, [jnp.zeros((L,), jnp.int32)])
            idx_flat[pl.ds(n, L)] = dup0

            def copy_body(r, _):
                idx2d[r, :] = idx_flat[pl.ds(r * L, L)]
                return 0

            lax.fori_loop(0, nb, copy_body, 0)
            pltpu.sync_copy(idx2d, idx_out.at[wid])

    return body(labels3, pointer)


def _sc_apply(out_ref_arg, tokens, idx_out, nb_out):
    mesh = plsc.VectorSubcoreMesh(core_axis_name="c", subcore_axis_name="s")

    @pl.kernel(
        out_type=(),
        mesh=mesh,
        compiler_params=pltpu.CompilerParams(needs_layout_passes=False),
        scratch_types=[
            pltpu.VMEM((MAXB, L), jnp.int32),
            pltpu.VMEM((L,), jnp.int32),
            pltpu.VMEM((2, L, DIM), jnp.float32),
            pltpu.SemaphoreType.DMA((2,)),
            pltpu.SemaphoreType.DMA((2,)),
        ],
    )
    def body(out_hbm, tokens_hbm, idx_hbm, nb_hbm,
             idx2d, nb_v, tok_v, gsem, ssem):
        wid = lax.axis_index("s") * NC + lax.axis_index("c")
        pltpu.sync_copy(nb_hbm.at[wid], nb_v)
        nb = jnp.max(nb_v[...])

        @pl.when(nb > 0)
        def _apply():
            pltpu.sync_copy(idx_hbm.at[wid], idx2d)

            def batch_body(b, _):
                s = b % 2

                @pl.when(b >= 2)
                def _reclaim():
                    pltpu.make_async_copy(
                        tok_v.at[s], out_hbm.at[idx2d.at[b]],
                        ssem.at[s]).wait()

                pltpu.make_async_copy(
                    tokens_hbm.at[idx2d.at[b]], tok_v.at[s],
                    gsem.at[s]).start()

                @pl.when(b >= 1)
                def _retire():
                    sp = (b - 1) % 2
                    pltpu.make_async_copy(
                        tokens_hbm.at[idx2d.at[b - 1]], tok_v.at[sp],
                        gsem.at[sp]).wait()
                    pltpu.make_async_copy(
                        tok_v.at[sp], out_hbm.at[idx2d.at[b - 1]],
                        ssem.at[sp]).start()
                return 0

            lax.fori_loop(0, nb, batch_body, 0)

            sl = (nb - 1) % 2
            pltpu.make_async_copy(
                tokens_hbm.at[idx2d.at[nb - 1]], tok_v.at[sl],
                gsem.at[sl]).wait()
            pltpu.make_async_copy(
                tok_v.at[sl], out_hbm.at[idx2d.at[nb - 1]],
                ssem.at[sl]).start()
            for s in range(2):
                @pl.when(nb > s)
                def _drain(s=s):
                    pltpu.make_async_copy(
                        tok_v.at[s], out_hbm.at[idx2d.at[0]],
                        ssem.at[s]).wait()

    body(out_ref_arg, tokens, idx_out, nb_out)


def kernel(tokens, token_labels, buffer, pointer):
    labels = token_labels.astype(jnp.int32)
    table_pad = jnp.zeros((PADC, DIM), jnp.float32).at[:NUM_CLASSES].set(
        buffer[:, 0, :])
    idx_out, nb_out = _sc_scan(labels.reshape(NW, GROUPS, L),
                               pointer.astype(jnp.int32))
    dense = _tc_expand(labels.reshape(TC_GRID, 1, TC_BLK), table_pad)
    out_ref = jax.new_ref(dense)
    _sc_apply(out_ref, tokens, idx_out, nb_out)
    return jax.freeze(out_ref)


# speedup vs baseline: 1.0325x; 1.0014x over previous
import jax
import jax.numpy as jnp
from jax import lax
from jax.experimental import pallas as pl
from jax.experimental.pallas import tpu as pltpu
from jax.experimental.pallas import tpu_sc as plsc

NUM_CLASSES = 80
DIM = 256
N_TOKENS = 131072

NC = 2
NS = 16
L = 16
NW = NC * NS

TOK_PER_W = N_TOKENS // NW
GROUPS = TOK_PER_W // L
MAXB = GROUPS + 1

TC_BLK = 8192
TC_GRID = N_TOKENS // TC_BLK
PADC = 128


def _tc_expand(labels3, table_pad):
    def body(lab_ref, tab_ref, out_ref):
        lab = lab_ref[0, 0, :]
        onehot = (lab[:, None] == lax.broadcasted_iota(
            jnp.int32, (TC_BLK, PADC), 1)).astype(jnp.float32)
        out_ref[...] = jnp.dot(onehot, tab_ref[...],
                               preferred_element_type=jnp.float32)

    return pl.pallas_call(
        body,
        grid=(TC_GRID,),
        in_specs=[
            pl.BlockSpec((1, 1, TC_BLK), lambda i: (i, 0, 0)),
            pl.BlockSpec((PADC, DIM), lambda i: (0, 0)),
        ],
        out_specs=pl.BlockSpec((TC_BLK, DIM), lambda i: (i, 0)),
        out_shape=jax.ShapeDtypeStruct((N_TOKENS, DIM), jnp.float32),
        compiler_params=pltpu.CompilerParams(
            dimension_semantics=("arbitrary",)),
    )(labels3, table_pad)


def _sc_scan(labels3, pointer):
    mesh = plsc.VectorSubcoreMesh(core_axis_name="c", subcore_axis_name="s")

    @pl.kernel(
        out_type=(
            jax.ShapeDtypeStruct((NW, MAXB, L), jnp.int32),
            jax.ShapeDtypeStruct((NW, L), jnp.int32),
        ),
        mesh=mesh,
        compiler_params=pltpu.CompilerParams(needs_layout_passes=False),
        scratch_types=[
            pltpu.VMEM((GROUPS, L), jnp.int32),
            pltpu.VMEM((NUM_CLASSES,), jnp.int32),
            pltpu.VMEM((TOK_PER_W + L,), jnp.int32),
            pltpu.VMEM((MAXB, L), jnp.int32),
            pltpu.VMEM((L,), jnp.int32),
        ],
    )
    def body(labels_hbm, ptr_hbm, idx_out, nb_out,
             lab_v, ptr_v, idx_flat, idx2d, nb_v):
        wid = lax.axis_index("s") * NC + lax.axis_index("c")
        pltpu.sync_copy(labels_hbm.at[wid], lab_v)
        pltpu.sync_copy(ptr_hbm, ptr_v)
        base = wid * TOK_PER_W
        iota = lax.iota(jnp.int32, L)

        def scan_body(q, n):
            invs, gidxs, cnts = [], [], []
            for j in range(4):
                g = q * 4 + j
                inv = plsc.load_gather(ptr_v, [lab_v[g, :]]) == 0
                invs.append(inv)
                gidxs.append(jnp.full((L,), base + g * L, jnp.int32) + iota)
                cnts.append(jnp.sum(inv.astype(jnp.int32)))
            off = n
            for j in range(4):
                plsc.store_compressed(
                    idx_flat.at[pl.ds(off, L)], gidxs[j], mask=invs[j])
                off = off + cnts[j]
            return off

        n = lax.fori_loop(0, GROUPS // 4, scan_body, 0)
        nb = (n + (L - 1)) >> 4
        nb_v[...] = jnp.full((L,), nb, jnp.int32)
        pltpu.sync_copy(nb_v, nb_out.at[wid])

        @pl.when(n > 0)
        def _emit():
            dup0 = plsc.load_gather(idx_flat, [jnp.zeros((L,), jnp.int32)])
            idx_flat[pl.ds(n, L)] = dup0

            def copy_body(r, _):
                idx2d[r, :] = idx_flat[pl.ds(r * L, L)]
                return 0

            lax.fori_loop(0, nb, copy_body, 0)
            pltpu.sync_copy(idx2d, idx_out.at[wid])

    return body(labels3, pointer)


def _sc_apply(out_ref_arg, tokens, idx_out, nb_out):
    mesh = plsc.VectorSubcoreMesh(core_axis_name="c", subcore_axis_name="s")

    @pl.kernel(
        out_type=(),
        mesh=mesh,
        compiler_params=pltpu.CompilerParams(needs_layout_passes=False),
        scratch_types=[
            pltpu.VMEM((MAXB, L), jnp.int32),
            pltpu.VMEM((L,), jnp.int32),
            pltpu.VMEM((2, L, DIM), jnp.float32),
            pltpu.SemaphoreType.DMA((2,)),
            pltpu.SemaphoreType.DMA((2,)),
        ],
    )
    def body(out_hbm, tokens_hbm, idx_hbm, nb_hbm,
             idx2d, nb_v, tok_v, gsem, ssem):
        wid = lax.axis_index("s") * NC + lax.axis_index("c")
        pltpu.sync_copy(nb_hbm.at[wid], nb_v)
        nb = jnp.max(nb_v[...])

        @pl.when(nb > 0)
        def _apply():
            pltpu.sync_copy(idx_hbm.at[wid], idx2d)

            def batch_body(b, _):
                s = b % 2

                @pl.when(b >= 2)
                def _reclaim():
                    pltpu.make_async_copy(
                        tok_v.at[s], out_hbm.at[idx2d.at[b]],
                        ssem.at[s]).wait()

                pltpu.make_async_copy(
                    tokens_hbm.at[idx2d.at[b]], tok_v.at[s],
                    gsem.at[s]).start()

                @pl.when(b >= 1)
                def _retire():
                    sp = (b - 1) % 2
                    pltpu.make_async_copy(
                        tokens_hbm.at[idx2d.at[b - 1]], tok_v.at[sp],
                        gsem.at[sp]).wait()
                    pltpu.make_async_copy(
                        tok_v.at[sp], out_hbm.at[idx2d.at[b - 1]],
                        ssem.at[sp]).start()
                return 0

            lax.fori_loop(0, nb, batch_body, 0)

            sl = (nb - 1) % 2
            pltpu.make_async_copy(
                tokens_hbm.at[idx2d.at[nb - 1]], tok_v.at[sl],
                gsem.at[sl]).wait()
            pltpu.make_async_copy(
                tok_v.at[sl], out_hbm.at[idx2d.at[nb - 1]],
                ssem.at[sl]).start()
            for s in range(2):
                @pl.when(nb > s)
                def _drain(s=s):
                    pltpu.make_async_copy(
                        tok_v.at[s], out_hbm.at[idx2d.at[0]],
                        ssem.at[s]).wait()

    body(out_ref_arg, tokens, idx_out, nb_out)


def kernel(tokens, token_labels, buffer, pointer):
    labels = token_labels.astype(jnp.int32)
    table_pad = jnp.zeros((PADC, DIM), jnp.float32).at[:NUM_CLASSES].set(
        buffer[:, 0, :])
    idx_out, nb_out = _sc_scan(labels.reshape(NW, GROUPS, L),
                               pointer.astype(jnp.int32))
    dense = _tc_expand(labels.reshape(TC_GRID, 1, TC_BLK), table_pad)
    out_ref = jax.new_ref(dense)
    _sc_apply(out_ref, tokens, idx_out, nb_out)
    return jax.freeze(out_ref)
